# Initial kernel scaffold; baseline (speedup 1.0000x reference)
import jax
import jax.numpy as jnp
from jax.experimental import pallas as pl


def kernel(compressed_tokens, original_tokens, projections):
    raise NotImplementedError("write your pallas kernel here")


# speedup vs baseline: 3.7405x; 3.7405x over previous
import functools

import jax
import jax.numpy as jnp
from jax.experimental import pallas as pl
from jax.experimental.pallas import tpu as pltpu

_B, _N1, _N2, _D, _P = 8, 2048, 8192, 128, 64
_BPS = 2
_C = _BPS * _P
_STEPS = _B // _BPS


def _bitonic_sort_ref(buf):
    n, c = buf.shape
    levels = n.bit_length() - 1
    for s in range(1, levels + 1):
        for j in range(s - 1, -1, -1):
            d = 1 << j
            g = n // (2 * d)
            x = buf[...].reshape(g, 2, d, c)
            a, b = x[:, 0], x[:, 1]
            mn = jnp.minimum(a, b)
            mx = jnp.maximum(a, b)
            if s == levels:
                lo, hi = mn, mx
            else:
                gi = jax.lax.broadcasted_iota(jnp.int32, (g, 1, 1), 0)
                desc = ((gi >> (s - 1 - j)) & 1) == 1
                lo = jnp.where(desc, mx, mn)
                hi = jnp.where(desc, mn, mx)
            buf[...] = jnp.concatenate(
                [lo[:, None], hi[:, None]], axis=1).reshape(n, c)


def _proj_body(x_ref, y_ref, p_ref, xp_ref, yp_ref):
    pr = p_ref[...]
    pn = pr * jax.lax.rsqrt(jnp.sum(pr * pr, axis=1, keepdims=True))

    def project(tokens):
        return jax.lax.dot_general(
            tokens, pn, (((1,), (1,)), ((), ())),
            preferred_element_type=jnp.float32)

    xp_ref[...] = jnp.concatenate([project(x_ref[0]), project(x_ref[1])], axis=1)
    yp_ref[...] = jnp.concatenate([project(y_ref[0]), project(y_ref[1])], axis=1)


def _sort_body(xp_ref, yp_ref, coef_ref, o_ref, xbuf, ybuf):
    step = pl.program_id(0)

    xbuf[...] = xp_ref[...]
    ybuf[...] = yp_ref[...]
    _bitonic_sort_ref(xbuf)
    _bitonic_sort_ref(ybuf)

    y_re = ybuf[...].reshape(_N1, _N2 // _N1, _C)
    yi = jnp.zeros((_N1, _C), jnp.float32)
    for d in range(_N2 // _N1):
        yi = yi + coef_ref[:, d][:, None] * y_re[:, d, :]

    acc = jnp.sum(jnp.abs(xbuf[...] - yi))

    @pl.when(step == 0)
    def _():
        o_ref[...] = jnp.zeros((1, 1), jnp.float32)

    o_ref[...] += acc

    @pl.when(step == _STEPS - 1)
    def _():
        o_ref[...] = o_ref[...] * (1.0 / (_N1 * _B * _P))


@functools.partial(jax.jit, static_argnames=())
def kernel(compressed_tokens, original_tokens, projections):
    idx = jnp.linspace(0.0, _N2 - 1, _N1)
    fl = idx.astype(jnp.int32)
    ce = jnp.minimum(fl + 1, _N2 - 1)
    w = idx - fl.astype(jnp.float32)
    base = (_N2 // _N1) * jnp.arange(_N1, dtype=jnp.int32)
    dr = jnp.arange(_N2 // _N1, dtype=jnp.int32)[None, :]
    coef = ((1.0 - w)[:, None] * ((fl - base)[:, None] == dr)
            + w[:, None] * ((ce - base)[:, None] == dr)).astype(jnp.float32)
    coef = jnp.pad(coef, ((0, 0), (0, 128 - _N2 // _N1)))

    xp, yp = pl.pallas_call(
        _proj_body,
        grid=(_STEPS,),
        in_specs=[
            pl.BlockSpec((_BPS, _N1, _D), lambda j: (j, 0, 0)),
            pl.BlockSpec((_BPS, _N2, _D), lambda j: (j, 0, 0)),
            pl.BlockSpec((_P, _D), lambda j: (0, 0)),
        ],
        out_specs=[
            pl.BlockSpec((_N1, _C), lambda j: (0, j)),
            pl.BlockSpec((_N2, _C), lambda j: (0, j)),
        ],
        out_shape=[
            jax.ShapeDtypeStruct((_N1, _STEPS * _C), jnp.float32),
            jax.ShapeDtypeStruct((_N2, _STEPS * _C), jnp.float32),
        ],
    )(compressed_tokens, original_tokens, projections)

    out = pl.pallas_call(
        _sort_body,
        grid=(_STEPS,),
        in_specs=[
            pl.BlockSpec((_N1, _C), lambda j: (0, j)),
            pl.BlockSpec((_N2, _C), lambda j: (0, j)),
            pl.BlockSpec((_N1, 128), lambda j: (0, 0)),
        ],
        out_specs=pl.BlockSpec((1, 1), lambda j: (0, 0)),
        out_shape=jax.ShapeDtypeStruct((1, 1), jnp.float32),
        scratch_shapes=[
            pltpu.VMEM((_N1, _C), jnp.float32),
            pltpu.VMEM((_N2, _C), jnp.float32),
        ],
    )(xp, yp, coef)
    return out[0, 0]


# speedup vs baseline: 5.0661x; 1.3544x over previous
import functools

import jax
import jax.numpy as jnp
from jax.experimental import pallas as pl
from jax.experimental.pallas import tpu as pltpu

_B, _N1, _N2, _D, _P = 8, 2048, 8192, 128, 64
_BPS = 2
_C = _BPS * _P
_STEPS = _B // _BPS


def _bitonic_sort_ref(buf):
    n, c = buf.shape
    levels = n.bit_length() - 1
    for s in range(1, levels + 1):
        for j in range(s - 1, -1, -1):
            d = 1 << j
            if d < 8:
                v = buf[...].reshape(n // 8, 8, c)
                pd = jnp.roll(v, -d, axis=1)
                pu = jnp.roll(v, d, axis=1)
                ir = jax.lax.broadcasted_iota(jnp.int32, (1, 8, 1), 1)
                lower = (ir & d) == 0
                partner = jnp.where(lower, pd, pu)
                mn = jnp.minimum(v, partner)
                mx = jnp.maximum(v, partner)
                if s == levels:
                    keep_mn = lower
                elif s >= 3:
                    ib = jax.lax.broadcasted_iota(
                        jnp.int32, (n // 8, 1, 1), 0)
                    asc = ((ib >> (s - 3)) & 1) == 0
                    keep_mn = lower == asc
                else:
                    asc = ((ir >> s) & 1) == 0
                    keep_mn = lower == asc
                buf[...] = jnp.where(keep_mn, mn, mx).reshape(n, c)
            else:
                g = n // (2 * d)
                x = buf[...].reshape(g, 2, d, c)
                a, b = x[:, 0], x[:, 1]
                mn = jnp.minimum(a, b)
                mx = jnp.maximum(a, b)
                if s == levels:
                    lo, hi = mn, mx
                else:
                    gi = jax.lax.broadcasted_iota(jnp.int32, (g, 1, 1), 0)
                    desc = ((gi >> (s - 1 - j)) & 1) == 1
                    lo = jnp.where(desc, mx, mn)
                    hi = jnp.where(desc, mn, mx)
                buf[...] = jnp.concatenate(
                    [lo[:, None], hi[:, None]], axis=1).reshape(n, c)


def _proj_body(x_ref, y_ref, p_ref, xp_ref, yp_ref):
    pr = p_ref[...]
    pn = pr * jax.lax.rsqrt(jnp.sum(pr * pr, axis=1, keepdims=True))

    def project(tokens):
        return jax.lax.dot_general(
            tokens, pn, (((1,), (1,)), ((), ())),
            preferred_element_type=jnp.float32)

    xp_ref[...] = jnp.concatenate([project(x_ref[0]), project(x_ref[1])], axis=1)
    yp_ref[...] = jnp.concatenate([project(y_ref[0]), project(y_ref[1])], axis=1)


def _sort_body(xp_hbm, yp_hbm, coef_ref, o_ref, xbuf, ybuf, sem_x, sem_y):
    step = pl.program_id(0)

    cx = pltpu.make_async_copy(
        xp_hbm.at[:, pl.ds(step * _C, _C)], xbuf, sem_x)
    cy = pltpu.make_async_copy(
        yp_hbm.at[:, pl.ds(step * _C, _C)], ybuf, sem_y)
    cx.start()
    cy.start()
    cx.wait()
    cy.wait()
    _bitonic_sort_ref(xbuf)
    _bitonic_sort_ref(ybuf)

    y_re = ybuf[...].reshape(_N1, _N2 // _N1, _C)
    yi = jnp.zeros((_N1, _C), jnp.float32)
    for d in range(_N2 // _N1):
        yi = yi + coef_ref[:, d][:, None] * y_re[:, d, :]

    acc = jnp.sum(jnp.abs(xbuf[...] - yi))

    @pl.when(step == 0)
    def _():
        o_ref[...] = jnp.zeros((1, 1), jnp.float32)

    o_ref[...] += acc

    @pl.when(step == _STEPS - 1)
    def _():
        o_ref[...] = o_ref[...] * (1.0 / (_N1 * _B * _P))


@functools.partial(jax.jit, static_argnames=())
def kernel(compressed_tokens, original_tokens, projections):
    idx = jnp.linspace(0.0, _N2 - 1, _N1)
    fl = idx.astype(jnp.int32)
    ce = jnp.minimum(fl + 1, _N2 - 1)
    w = idx - fl.astype(jnp.float32)
    base = (_N2 // _N1) * jnp.arange(_N1, dtype=jnp.int32)
    dr = jnp.arange(_N2 // _N1, dtype=jnp.int32)[None, :]
    coef = ((1.0 - w)[:, None] * ((fl - base)[:, None] == dr)
            + w[:, None] * ((ce - base)[:, None] == dr)).astype(jnp.float32)
    coef = jnp.pad(coef, ((0, 0), (0, 128 - _N2 // _N1)))

    xp, yp = pl.pallas_call(
        _proj_body,
        grid=(_STEPS,),
        in_specs=[
            pl.BlockSpec((_BPS, _N1, _D), lambda j: (j, 0, 0)),
            pl.BlockSpec((_BPS, _N2, _D), lambda j: (j, 0, 0)),
            pl.BlockSpec((_P, _D), lambda j: (0, 0)),
        ],
        out_specs=[
            pl.BlockSpec((_N1, _C), lambda j: (0, j)),
            pl.BlockSpec((_N2, _C), lambda j: (0, j)),
        ],
        out_shape=[
            jax.ShapeDtypeStruct((_N1, _STEPS * _C), jnp.float32),
            jax.ShapeDtypeStruct((_N2, _STEPS * _C), jnp.float32),
        ],
    )(compressed_tokens, original_tokens, projections)

    out = pl.pallas_call(
        _sort_body,
        grid=(_STEPS,),
        in_specs=[
            pl.BlockSpec(memory_space=pl.ANY),
            pl.BlockSpec(memory_space=pl.ANY),
            pl.BlockSpec((_N1, 128), lambda j: (0, 0)),
        ],
        out_specs=pl.BlockSpec((1, 1), lambda j: (0, 0)),
        out_shape=jax.ShapeDtypeStruct((1, 1), jnp.float32),
        scratch_shapes=[
            pltpu.VMEM((_N1, _C), jnp.float32),
            pltpu.VMEM((_N2, _C), jnp.float32),
            pltpu.SemaphoreType.DMA,
            pltpu.SemaphoreType.DMA,
        ],
    )(xp, yp, coef)
    return out[0, 0]


# speedup vs baseline: 5.0664x; 1.0001x over previous
import functools

import jax
import jax.numpy as jnp
from jax.experimental import pallas as pl
from jax.experimental.pallas import tpu as pltpu

_B, _N1, _N2, _D, _P = 8, 2048, 8192, 128, 64
_BPS = 2
_C = _BPS * _P
_STEPS = _B // _BPS


def _bitonic_sort_ref(buf):
    n, c = buf.shape
    levels = n.bit_length() - 1
    for s in range(1, levels + 1):
        for j in range(s - 1, -1, -1):
            d = 1 << j
            if d < 8:
                v = buf[...].reshape(n // 8, 8, c)
                ir = jax.lax.broadcasted_iota(jnp.int32, (1, 8, 1), 1)
                lower = (ir & d) == 0
                if d == 4:
                    partner = pltpu.roll(v, 4, axis=1)
                else:
                    pd = pltpu.roll(v, 8 - d, axis=1)
                    pu = pltpu.roll(v, d, axis=1)
                    partner = jnp.where(lower, pd, pu)
                mn = jnp.minimum(v, partner)
                mx = jnp.maximum(v, partner)
                if s == levels:
                    keep_mn = lower
                elif s >= 3:
                    ib = jax.lax.broadcasted_iota(
                        jnp.int32, (n // 8, 1, 1), 0)
                    asc = ((ib >> (s - 3)) & 1) == 0
                    keep_mn = lower == asc
                else:
                    asc = ((ir >> s) & 1) == 0
                    keep_mn = lower == asc
                buf[...] = jnp.where(keep_mn, mn, mx).reshape(n, c)
            else:
                g = n // (2 * d)
                x = buf[...].reshape(g, 2, d, c)
                a, b = x[:, 0], x[:, 1]
                mn = jnp.minimum(a, b)
                mx = jnp.maximum(a, b)
                if s == levels:
                    lo, hi = mn, mx
                else:
                    gi = jax.lax.broadcasted_iota(jnp.int32, (g, 1, 1), 0)
                    desc = ((gi >> (s - 1 - j)) & 1) == 1
                    lo = jnp.where(desc, mx, mn)
                    hi = jnp.where(desc, mn, mx)
                buf[...] = jnp.concatenate(
                    [lo[:, None], hi[:, None]], axis=1).reshape(n, c)


def _proj_body(x_ref, y_ref, p_ref, xp_ref, yp_ref):
    pr = p_ref[...]
    pn = pr * jax.lax.rsqrt(jnp.sum(pr * pr, axis=1, keepdims=True))

    def project(tokens):
        return jax.lax.dot_general(
            tokens, pn, (((1,), (1,)), ((), ())),
            preferred_element_type=jnp.float32)

    xp_ref[...] = jnp.concatenate([project(x_ref[0]), project(x_ref[1])], axis=1)
    yp_ref[...] = jnp.concatenate([project(y_ref[0]), project(y_ref[1])], axis=1)


def _sort_body(xp_hbm, yp_hbm, coef_ref, o_ref, xbuf, ybuf, sem_x, sem_y):
    step = pl.program_id(0)

    cx = pltpu.make_async_copy(
        xp_hbm.at[:, pl.ds(step * _C, _C)], xbuf, sem_x)
    cy = pltpu.make_async_copy(
        yp_hbm.at[:, pl.ds(step * _C, _C)], ybuf, sem_y)
    cx.start()
    cy.start()
    cx.wait()
    cy.wait()
    _bitonic_sort_ref(xbuf)
    _bitonic_sort_ref(ybuf)

    y_re = ybuf[...].reshape(_N1, _N2 // _N1, _C)
    yi = jnp.zeros((_N1, _C), jnp.float32)
    for d in range(_N2 // _N1):
        yi = yi + coef_ref[:, d][:, None] * y_re[:, d, :]

    acc = jnp.sum(jnp.abs(xbuf[...] - yi))

    @pl.when(step == 0)
    def _():
        o_ref[...] = jnp.zeros((1, 1), jnp.float32)

    o_ref[...] += acc

    @pl.when(step == _STEPS - 1)
    def _():
        o_ref[...] = o_ref[...] * (1.0 / (_N1 * _B * _P))


@functools.partial(jax.jit, static_argnames=())
def kernel(compressed_tokens, original_tokens, projections):
    idx = jnp.linspace(0.0, _N2 - 1, _N1)
    fl = idx.astype(jnp.int32)
    ce = jnp.minimum(fl + 1, _N2 - 1)
    w = idx - fl.astype(jnp.float32)
    base = (_N2 // _N1) * jnp.arange(_N1, dtype=jnp.int32)
    dr = jnp.arange(_N2 // _N1, dtype=jnp.int32)[None, :]
    coef = ((1.0 - w)[:, None] * ((fl - base)[:, None] == dr)
            + w[:, None] * ((ce - base)[:, None] == dr)).astype(jnp.float32)
    coef = jnp.pad(coef, ((0, 0), (0, 128 - _N2 // _N1)))

    xp, yp = pl.pallas_call(
        _proj_body,
        grid=(_STEPS,),
        in_specs=[
            pl.BlockSpec((_BPS, _N1, _D), lambda j: (j, 0, 0)),
            pl.BlockSpec((_BPS, _N2, _D), lambda j: (j, 0, 0)),
            pl.BlockSpec((_P, _D), lambda j: (0, 0)),
        ],
        out_specs=[
            pl.BlockSpec((_N1, _C), lambda j: (0, j)),
            pl.BlockSpec((_N2, _C), lambda j: (0, j)),
        ],
        out_shape=[
            jax.ShapeDtypeStruct((_N1, _STEPS * _C), jnp.float32),
            jax.ShapeDtypeStruct((_N2, _STEPS * _C), jnp.float32),
        ],
    )(compressed_tokens, original_tokens, projections)

    out = pl.pallas_call(
        _sort_body,
        grid=(_STEPS,),
        in_specs=[
            pl.BlockSpec(memory_space=pl.ANY),
            pl.BlockSpec(memory_space=pl.ANY),
            pl.BlockSpec((_N1, 128), lambda j: (0, 0)),
        ],
        out_specs=pl.BlockSpec((1, 1), lambda j: (0, 0)),
        out_shape=jax.ShapeDtypeStruct((1, 1), jnp.float32),
        scratch_shapes=[
            pltpu.VMEM((_N1, _C), jnp.float32),
            pltpu.VMEM((_N2, _C), jnp.float32),
            pltpu.SemaphoreType.DMA,
            pltpu.SemaphoreType.DMA,
        ],
    )(xp, yp, coef)
    return out[0, 0]


# speedup vs baseline: 6.4354x; 1.2702x over previous
import functools

import jax
import jax.numpy as jnp
from jax.experimental import pallas as pl
from jax.experimental.pallas import tpu as pltpu

_B, _N1, _N2, _D, _P = 8, 2048, 8192, 128, 64
_BPS = 2
_C = _BPS * _P
_STEPS = _B // _BPS


def _bitonic_sort_ref(buf):
    n, c = buf.shape
    levels = n.bit_length() - 1
    ir = jax.lax.broadcasted_iota(jnp.int32, (1, 8, 1), 1)

    def ce_small(v, s, j):
        d = 1 << j
        lower = (ir & d) == 0
        if d == 4:
            partner = pltpu.roll(v, 4, axis=1)
        else:
            partner = jnp.where(lower, pltpu.roll(v, 8 - d, axis=1),
                                pltpu.roll(v, d, axis=1))
        mn = jnp.minimum(v, partner)
        mx = jnp.maximum(v, partner)
        if s == levels:
            keep = lower
        elif s >= 3:
            ib = jax.lax.broadcasted_iota(jnp.int32, (n // 8, 1, 1), 0)
            keep = lower == (((ib >> (s - 3)) & 1) == 0)
        else:
            keep = lower == (((ir >> s) & 1) == 0)
        return jnp.where(keep, mn, mx)

    v = buf[...].reshape(n // 8, 8, c)
    for s in range(1, min(3, levels) + 1):
        for j in range(s - 1, -1, -1):
            v = ce_small(v, s, j)
    buf[...] = v.reshape(n, c)

    for s in range(4, levels + 1):
        for j in range(s - 1, 2, -1):
            d = 1 << j
            g = n // (2 * d)
            x = buf[...].reshape(g, 2, d, c)
            a, b = x[:, 0], x[:, 1]
            mn = jnp.minimum(a, b)
            mx = jnp.maximum(a, b)
            if s == levels:
                lo, hi = mn, mx
            else:
                gi = jax.lax.broadcasted_iota(jnp.int32, (g, 1, 1), 0)
                desc = ((gi >> (s - 1 - j)) & 1) == 1
                lo = jnp.where(desc, mx, mn)
                hi = jnp.where(desc, mn, mx)
            buf[...] = jnp.concatenate(
                [lo[:, None], hi[:, None]], axis=1).reshape(n, c)
        v = buf[...].reshape(n // 8, 8, c)
        for j in (2, 1, 0):
            v = ce_small(v, s, j)
        buf[...] = v.reshape(n, c)


def _proj_body(x_ref, y_ref, p_ref, xp_ref, yp_ref):
    pr = p_ref[...]
    pn = pr * jax.lax.rsqrt(jnp.sum(pr * pr, axis=1, keepdims=True))

    def project(tokens):
        return jax.lax.dot_general(
            tokens, pn, (((1,), (1,)), ((), ())),
            preferred_element_type=jnp.float32)

    xp_ref[...] = jnp.concatenate([project(x_ref[0]), project(x_ref[1])], axis=1)
    yp_ref[...] = jnp.concatenate([project(y_ref[0]), project(y_ref[1])], axis=1)


def _sort_body(xp_hbm, yp_hbm, coef_ref, o_ref, xbuf, ybuf, sem_x, sem_y):
    step = pl.program_id(0)

    cx = pltpu.make_async_copy(
        xp_hbm.at[:, pl.ds(step * _C, _C)], xbuf, sem_x)
    cy = pltpu.make_async_copy(
        yp_hbm.at[:, pl.ds(step * _C, _C)], ybuf, sem_y)
    cx.start()
    cy.start()
    cx.wait()
    cy.wait()
    _bitonic_sort_ref(xbuf)
    _bitonic_sort_ref(ybuf)

    y_re = ybuf[...].reshape(_N1, _N2 // _N1, _C)
    yi = jnp.zeros((_N1, _C), jnp.float32)
    for d in range(_N2 // _N1):
        yi = yi + coef_ref[:, d][:, None] * y_re[:, d, :]

    acc = jnp.sum(jnp.abs(xbuf[...] - yi))

    @pl.when(step == 0)
    def _():
        o_ref[...] = jnp.zeros((1, 1), jnp.float32)

    o_ref[...] += acc

    @pl.when(step == _STEPS - 1)
    def _():
        o_ref[...] = o_ref[...] * (1.0 / (_N1 * _B * _P))


@functools.partial(jax.jit, static_argnames=())
def kernel(compressed_tokens, original_tokens, projections):
    idx = jnp.linspace(0.0, _N2 - 1, _N1)
    fl = idx.astype(jnp.int32)
    ce = jnp.minimum(fl + 1, _N2 - 1)
    w = idx - fl.astype(jnp.float32)
    base = (_N2 // _N1) * jnp.arange(_N1, dtype=jnp.int32)
    dr = jnp.arange(_N2 // _N1, dtype=jnp.int32)[None, :]
    coef = ((1.0 - w)[:, None] * ((fl - base)[:, None] == dr)
            + w[:, None] * ((ce - base)[:, None] == dr)).astype(jnp.float32)
    coef = jnp.pad(coef, ((0, 0), (0, 128 - _N2 // _N1)))

    xp, yp = pl.pallas_call(
        _proj_body,
        grid=(_STEPS,),
        in_specs=[
            pl.BlockSpec((_BPS, _N1, _D), lambda j: (j, 0, 0)),
            pl.BlockSpec((_BPS, _N2, _D), lambda j: (j, 0, 0)),
            pl.BlockSpec((_P, _D), lambda j: (0, 0)),
        ],
        out_specs=[
            pl.BlockSpec((_N1, _C), lambda j: (0, j)),
            pl.BlockSpec((_N2, _C), lambda j: (0, j)),
        ],
        out_shape=[
            jax.ShapeDtypeStruct((_N1, _STEPS * _C), jnp.float32),
            jax.ShapeDtypeStruct((_N2, _STEPS * _C), jnp.float32),
        ],
    )(compressed_tokens, original_tokens, projections)

    out = pl.pallas_call(
        _sort_body,
        grid=(_STEPS,),
        in_specs=[
            pl.BlockSpec(memory_space=pl.ANY),
            pl.BlockSpec(memory_space=pl.ANY),
            pl.BlockSpec((_N1, 128), lambda j: (0, 0)),
        ],
        out_specs=pl.BlockSpec((1, 1), lambda j: (0, 0)),
        out_shape=jax.ShapeDtypeStruct((1, 1), jnp.float32),
        scratch_shapes=[
            pltpu.VMEM((_N1, _C), jnp.float32),
            pltpu.VMEM((_N2, _C), jnp.float32),
            pltpu.SemaphoreType.DMA,
            pltpu.SemaphoreType.DMA,
        ],
    )(xp, yp, coef)
    return out[0, 0]
